# baseline (speedup 1.0000x reference)
import functools

import jax
import jax.numpy as jnp
from jax import lax
from jax.experimental import pallas as pl
from jax.experimental.pallas import tpu as pltpu
from jax.experimental.pallas import tpu_sc as plsc

V, F, P = 4096, 8192, 8192
NW = 32
FPW = F // NW
L = 16

PB = 16
FB = 512
NPB = P // PB
NFB = F // FB


def _sc_gather_body(verts_hbm, faces_hbm, out_hbm,
                    verts_v, idx0_v, idx1_v, idx2_v, out_v):
    c = lax.axis_index("c")
    s = lax.axis_index("s")
    w = s * 2 + c
    base = w * FPW
    pltpu.sync_copy(verts_hbm, verts_v)
    pltpu.sync_copy(faces_hbm.at[pl.ds(0 * F + base, FPW)], idx0_v)
    pltpu.sync_copy(faces_hbm.at[pl.ds(1 * F + base, FPW)], idx1_v)
    pltpu.sync_copy(faces_hbm.at[pl.ds(2 * F + base, FPW)], idx2_v)
    idxs = (idx0_v, idx1_v, idx2_v)
    for i in range(FPW // L):
        for slot in range(3):
            r = idxs[slot][pl.ds(i * L, L)]
            fi = r * 3
            for k in range(3):
                val = plsc.load_gather(verts_v, [fi + k])
                out_v[pl.ds((3 * slot + k) * FPW + i * L, L)] = val
    pltpu.sync_copy(out_v, out_hbm.at[pl.ds(w * 9 * FPW, 9 * FPW)])


@functools.partial(jax.jit, static_argnames=())
def _sc_gather(verts_flat, faces_flat):
    mesh = plsc.VectorSubcoreMesh(core_axis_name="c", subcore_axis_name="s")
    fn = pl.kernel(
        _sc_gather_body,
        out_type=jax.ShapeDtypeStruct((NW * 9 * FPW,), jnp.float32),
        mesh=mesh,
        compiler_params=pltpu.CompilerParams(needs_layout_passes=False),
        scratch_types=[
            pltpu.VMEM((3 * V,), jnp.float32),
            pltpu.VMEM((FPW,), jnp.int32),
            pltpu.VMEM((FPW,), jnp.int32),
            pltpu.VMEM((FPW,), jnp.int32),
            pltpu.VMEM((9 * FPW,), jnp.float32),
        ],
    )
    return fn(verts_flat, faces_flat)


def _safe_div(n, d):
    return n / jnp.where(jnp.abs(d) < 1e-12, 1.0, d)


def _tc_body(pts_ref, abc_ref, dist_ref, assoc_ref, loss_ref,
             rmin_ref, ridx_ref, acc_ref):
    pi = pl.program_id(0)
    fj = pl.program_id(1)

    @pl.when(fj == 0)
    def _():
        rmin_ref[...] = jnp.full((PB, 1), jnp.inf, jnp.float32)
        ridx_ref[...] = jnp.zeros((PB, 1), jnp.int32)

    pts = pts_ref[...]
    px = pts[:, 0:1]
    py = pts[:, 1:2]
    pz = pts[:, 2:3]

    ax = abc_ref[0:1, :]
    ay = abc_ref[1:2, :]
    az = abc_ref[2:3, :]
    bx = abc_ref[3:4, :]
    by = abc_ref[4:5, :]
    bz = abc_ref[5:6, :]
    cx = abc_ref[6:7, :]
    cy = abc_ref[7:8, :]
    cz = abc_ref[8:9, :]

    abx = bx - ax
    aby = by - ay
    abz = bz - az
    acx = cx - ax
    acy = cy - ay
    acz = cz - az

    apx = px - ax
    apy = py - ay
    apz = pz - az
    d1 = abx * apx + aby * apy + abz * apz
    d2 = acx * apx + acy * apy + acz * apz
    bpx = px - bx
    bpy = py - by
    bpz = pz - bz
    d3 = abx * bpx + aby * bpy + abz * bpz
    d4 = acx * bpx + acy * bpy + acz * bpz
    cpx = px - cx
    cpy = py - cy
    cpz = pz - cz
    d5 = abx * cpx + aby * cpy + abz * cpz
    d6 = acx * cpx + acy * cpy + acz * cpz

    vc = d1 * d4 - d3 * d2
    vb = d5 * d2 - d1 * d6
    va = d3 * d6 - d5 * d4
    v_ab = _safe_div(d1, d1 - d3)
    w_ac = _safe_div(d2, d2 - d6)
    d43 = d4 - d3
    d56 = d5 - d6
    w_bc = _safe_div(d43, d43 + d56)
    denom = _safe_div(jnp.ones_like(va), va + vb + vc)
    v_in = vb * denom
    w_in = vc * denom

    clx = ax + abx * v_in + acx * w_in
    cly = ay + aby * v_in + acy * w_in
    clz = az + abz * v_in + acz * w_in

    m_bc = (va <= 0) & (d43 >= 0) & (d56 >= 0)
    clx = jnp.where(m_bc, bx + (cx - bx) * w_bc, clx)
    cly = jnp.where(m_bc, by + (cy - by) * w_bc, cly)
    clz = jnp.where(m_bc, bz + (cz - bz) * w_bc, clz)
    m_ac = (vb <= 0) & (d2 >= 0) & (d6 <= 0)
    clx = jnp.where(m_ac, ax + acx * w_ac, clx)
    cly = jnp.where(m_ac, ay + acy * w_ac, cly)
    clz = jnp.where(m_ac, az + acz * w_ac, clz)
    m_ab = (vc <= 0) & (d1 >= 0) & (d3 <= 0)
    clx = jnp.where(m_ab, ax + abx * v_ab, clx)
    cly = jnp.where(m_ab, ay + aby * v_ab, cly)
    clz = jnp.where(m_ab, az + abz * v_ab, clz)
    m_c = (d6 >= 0) & (d5 <= d6)
    clx = jnp.where(m_c, cx, clx)
    cly = jnp.where(m_c, cy, cly)
    clz = jnp.where(m_c, cz, clz)
    m_b = (d3 >= 0) & (d4 <= d3)
    clx = jnp.where(m_b, bx, clx)
    cly = jnp.where(m_b, by, cly)
    clz = jnp.where(m_b, bz, clz)
    m_a = (d1 <= 0) & (d2 <= 0)
    clx = jnp.where(m_a, ax, clx)
    cly = jnp.where(m_a, ay, cly)
    clz = jnp.where(m_a, az, clz)

    dx = px - clx
    dy = py - cly
    dz = pz - clz
    sq = dx * dx + dy * dy + dz * dz

    bmin = jnp.min(sq, axis=1, keepdims=True)
    lane = lax.broadcasted_iota(jnp.int32, (PB, FB), 1) + fj * FB
    cand = jnp.min(jnp.where(sq == bmin, lane, jnp.int32(2**31 - 1)),
                   axis=1, keepdims=True)
    better = bmin < rmin_ref[...]
    rmin_ref[...] = jnp.where(better, bmin, rmin_ref[...])
    ridx_ref[...] = jnp.where(better, cand, ridx_ref[...])

    @pl.when(fj == NFB - 1)
    def _():
        dist_ref[...] = rmin_ref[...]
        assoc_ref[...] = ridx_ref[...]

        @pl.when(pi == 0)
        def _():
            acc_ref[0] = 0.0

        acc_ref[0] += jnp.sum(rmin_ref[...])

        @pl.when(pi == NPB - 1)
        def _():
            loss_ref[0, 0] = acc_ref[0] * (1.0 / P)


def _tc_sweep(points, abc9, interpret=False):
    return pl.pallas_call(
        _tc_body,
        grid=(NPB, NFB),
        in_specs=[
            pl.BlockSpec((PB, 3), lambda i, j: (i, 0)),
            pl.BlockSpec((9, FB), lambda i, j: (0, j)),
        ],
        out_specs=[
            pl.BlockSpec((PB, 1), lambda i, j: (i, 0)),
            pl.BlockSpec((PB, 1), lambda i, j: (i, 0)),
            pl.BlockSpec(memory_space=pltpu.SMEM),
        ],
        out_shape=[
            jax.ShapeDtypeStruct((P, 1), jnp.float32),
            jax.ShapeDtypeStruct((P, 1), jnp.int32),
            jax.ShapeDtypeStruct((1, 1), jnp.float32),
        ],
        scratch_shapes=[
            pltpu.VMEM((PB, 1), jnp.float32),
            pltpu.VMEM((PB, 1), jnp.int32),
            pltpu.SMEM((1,), jnp.float32),
        ],
        compiler_params=pltpu.CompilerParams(
            dimension_semantics=("arbitrary", "arbitrary"),
        ),
        interpret=interpret,
    )(points, abc9)


def kernel(verts, faces, points):
    faces_flat = faces.T.reshape(-1)
    abc_flat = _sc_gather(verts.reshape(-1), faces_flat)
    abc9 = abc_flat.reshape(NW, 9, FPW).transpose(1, 0, 2).reshape(9, F)
    dist2, assoc2, loss2 = _tc_sweep(points, abc9)
    return loss2[0, 0], dist2.reshape(-1), assoc2.reshape(-1)


# speedup vs baseline: 2.9496x; 2.9496x over previous
import functools

import jax
import jax.numpy as jnp
from jax import lax
from jax.experimental import pallas as pl
from jax.experimental.pallas import tpu as pltpu
from jax.experimental.pallas import tpu_sc as plsc

V, F, P = 4096, 8192, 8192
NW = 32
FPW = F // NW
L = 16

PSC = 2560
PTC = P - PSC

PB = 512
FB = 512
NPB = PTC // PB
NFB = F // FB

PPW = PSC // NW
INT_MAX = 2**31 - 1


def _safe_div(n, d):
    return n / jnp.where(jnp.abs(d) < 1e-12, 1.0, d)


def _tri_sq(px, py, pz, ax, ay, az, bx, by, bz, cx, cy, cz):
    abx = bx - ax
    aby = by - ay
    abz = bz - az
    acx = cx - ax
    acy = cy - ay
    acz = cz - az

    apx = px - ax
    apy = py - ay
    apz = pz - az
    d1 = abx * apx + aby * apy + abz * apz
    d2 = acx * apx + acy * apy + acz * apz
    bpx = px - bx
    bpy = py - by
    bpz = pz - bz
    d3 = abx * bpx + aby * bpy + abz * bpz
    d4 = acx * bpx + acy * bpy + acz * bpz
    cpx = px - cx
    cpy = py - cy
    cpz = pz - cz
    d5 = abx * cpx + aby * cpy + abz * cpz
    d6 = acx * cpx + acy * cpy + acz * cpz

    vc = d1 * d4 - d3 * d2
    vb = d5 * d2 - d1 * d6
    va = d3 * d6 - d5 * d4
    v_ab = _safe_div(d1, d1 - d3)
    w_ac = _safe_div(d2, d2 - d6)
    d43 = d4 - d3
    d56 = d5 - d6
    w_bc = _safe_div(d43, d43 + d56)
    denom = _safe_div(jnp.ones_like(va), va + vb + vc)
    v_in = vb * denom
    w_in = vc * denom

    clx = ax + abx * v_in + acx * w_in
    cly = ay + aby * v_in + acy * w_in
    clz = az + abz * v_in + acz * w_in

    m_bc = (va <= 0) & (d43 >= 0) & (d56 >= 0)
    clx = jnp.where(m_bc, bx + (cx - bx) * w_bc, clx)
    cly = jnp.where(m_bc, by + (cy - by) * w_bc, cly)
    clz = jnp.where(m_bc, bz + (cz - bz) * w_bc, clz)
    m_ac = (vb <= 0) & (d2 >= 0) & (d6 <= 0)
    clx = jnp.where(m_ac, ax + acx * w_ac, clx)
    cly = jnp.where(m_ac, ay + acy * w_ac, cly)
    clz = jnp.where(m_ac, az + acz * w_ac, clz)
    m_ab = (vc <= 0) & (d1 >= 0) & (d3 <= 0)
    clx = jnp.where(m_ab, ax + abx * v_ab, clx)
    cly = jnp.where(m_ab, ay + aby * v_ab, cly)
    clz = jnp.where(m_ab, az + abz * v_ab, clz)
    m_c = (d6 >= 0) & (d5 <= d6)
    clx = jnp.where(m_c, cx, clx)
    cly = jnp.where(m_c, cy, cly)
    clz = jnp.where(m_c, cz, clz)
    m_b = (d3 >= 0) & (d4 <= d3)
    clx = jnp.where(m_b, bx, clx)
    cly = jnp.where(m_b, by, cly)
    clz = jnp.where(m_b, bz, clz)
    m_a = (d1 <= 0) & (d2 <= 0)
    clx = jnp.where(m_a, ax, clx)
    cly = jnp.where(m_a, ay, cly)
    clz = jnp.where(m_a, az, clz)

    dx = px - clx
    dy = py - cly
    dz = pz - clz
    return dx * dx + dy * dy + dz * dz


def _sc_gather_body(verts_hbm, faces_hbm, out_hbm, verts_v, fidx_v, out_v):
    c = lax.axis_index("c")
    s = lax.axis_index("s")
    w = s * 2 + c
    base = w * FPW
    pltpu.sync_copy(verts_hbm, verts_v)
    pltpu.sync_copy(faces_hbm.at[pl.ds(3 * base, 3 * FPW)], fidx_v)
    iota3 = lax.iota(jnp.int32, L) * 3
    for i in range(FPW // L):
        for slot in range(3):
            r = plsc.load_gather(fidx_v, [iota3 + (48 * i + slot)])
            fi = r * 3
            for k in range(3):
                val = plsc.load_gather(verts_v, [fi + k])
                out_v[pl.ds((3 * slot + k) * FPW + i * L, L)] = val
    for row in range(9):
        pltpu.sync_copy(out_v.at[pl.ds(row * FPW, FPW)],
                        out_hbm.at[pl.ds(row * F + base, FPW)])


def _sc_gather(verts_flat, faces_flat):
    mesh = plsc.VectorSubcoreMesh(core_axis_name="c", subcore_axis_name="s")
    fn = pl.kernel(
        _sc_gather_body,
        out_type=jax.ShapeDtypeStruct((9 * F,), jnp.float32),
        mesh=mesh,
        compiler_params=pltpu.CompilerParams(needs_layout_passes=False),
        scratch_types=[
            pltpu.VMEM((3 * V,), jnp.float32),
            pltpu.VMEM((3 * FPW,), jnp.int32),
            pltpu.VMEM((9 * FPW,), jnp.float32),
        ],
    )
    return fn(verts_flat, faces_flat)


def _sc_sweep_body(abc_hbm, pts_hbm, dist_hbm, idx_hbm,
                   abc_v, pts_v, dist_v, idx_v):
    c = lax.axis_index("c")
    s = lax.axis_index("s")
    w = s * 2 + c
    pltpu.sync_copy(abc_hbm, abc_v)
    pltpu.sync_copy(pts_hbm.at[pl.ds(w * PPW * 3, PPW * 3)], pts_v)

    iota3 = lax.iota(jnp.int32, L) * 3

    def point_body(g, _):
        pbase = g * (L * 3)
        px = plsc.load_gather(pts_v, [iota3 + (pbase + 0)])
        py = plsc.load_gather(pts_v, [iota3 + (pbase + 1)])
        pz = plsc.load_gather(pts_v, [iota3 + (pbase + 2)])

        def blk_body(fb, carry):
            del fb
            rmin, ridx, bvec = carry
            for j in range(L):
                comps = [plsc.load_gather(abc_v, [bvec + (r * F + j)])
                         for r in range(9)]
                sq = _tri_sq(px, py, pz, *comps)
                better = sq < rmin
                rmin = jnp.where(better, sq, rmin)
                ridx = jnp.where(better, bvec + j, ridx)
            return rmin, ridx, bvec + L

        rmin0 = jnp.full((L,), jnp.inf, jnp.float32)
        ridx0 = jnp.zeros((L,), jnp.int32)
        bvec0 = jnp.zeros((L,), jnp.int32)
        rmin, ridx, _ = lax.fori_loop(0, F // L, blk_body,
                                      (rmin0, ridx0, bvec0))
        dist_v[pl.ds(g * L, L)] = rmin
        idx_v[pl.ds(g * L, L)] = ridx
        return 0

    lax.fori_loop(0, PPW // L, point_body, 0)

    pltpu.sync_copy(dist_v, dist_hbm.at[pl.ds(w * PPW, PPW)])
    pltpu.sync_copy(idx_v, idx_hbm.at[pl.ds(w * PPW, PPW)])


def _sc_sweep(abc_flat, pts_sc_flat):
    mesh = plsc.VectorSubcoreMesh(core_axis_name="c", subcore_axis_name="s")
    fn = pl.kernel(
        _sc_sweep_body,
        out_type=(jax.ShapeDtypeStruct((PSC,), jnp.float32),
                  jax.ShapeDtypeStruct((PSC,), jnp.int32)),
        mesh=mesh,
        compiler_params=pltpu.CompilerParams(needs_layout_passes=False),
        scratch_types=[
            pltpu.VMEM((9 * F,), jnp.float32),
            pltpu.VMEM((PPW * 3,), jnp.float32),
            pltpu.VMEM((PPW,), jnp.float32),
            pltpu.VMEM((PPW,), jnp.int32),
        ],
    )
    return fn(abc_flat, pts_sc_flat)


def _tc_body(pts_ref, abc_ref, dist_ref, assoc_ref, sum_ref,
             rmin_ref, ridx_ref, acc_ref):
    pi = pl.program_id(0)
    fj = pl.program_id(1)

    @pl.when(fj == 0)
    def _():
        rmin_ref[...] = jnp.full((PB, 1), jnp.inf, jnp.float32)
        ridx_ref[...] = jnp.zeros((PB, 1), jnp.int32)

    pts = pts_ref[...]
    px = pts[:, 0:1]
    py = pts[:, 1:2]
    pz = pts[:, 2:3]

    comps = [abc_ref[r:r + 1, :] for r in range(9)]
    sq = _tri_sq(px, py, pz, *comps)

    bmin = jnp.min(sq, axis=1, keepdims=True)
    lane = lax.broadcasted_iota(jnp.int32, (PB, FB), 1) + fj * FB
    cand = jnp.min(jnp.where(sq == bmin, lane, jnp.int32(INT_MAX)),
                   axis=1, keepdims=True)
    better = bmin < rmin_ref[...]
    rmin_ref[...] = jnp.where(better, bmin, rmin_ref[...])
    ridx_ref[...] = jnp.where(better, cand, ridx_ref[...])

    @pl.when(fj == NFB - 1)
    def _():
        dist_ref[...] = rmin_ref[...]
        assoc_ref[...] = ridx_ref[...]

        @pl.when(pi == 0)
        def _():
            acc_ref[0] = 0.0

        acc_ref[0] += jnp.sum(rmin_ref[...])

        @pl.when(pi == NPB - 1)
        def _():
            sum_ref[0, 0] = acc_ref[0]


def _tc_sweep(points_tc, abc9, interpret=False):
    return pl.pallas_call(
        _tc_body,
        grid=(NPB, NFB),
        in_specs=[
            pl.BlockSpec((PB, 3), lambda i, j: (i, 0)),
            pl.BlockSpec((9, FB), lambda i, j: (0, j)),
        ],
        out_specs=[
            pl.BlockSpec((PB, 1), lambda i, j: (i, 0)),
            pl.BlockSpec((PB, 1), lambda i, j: (i, 0)),
            pl.BlockSpec(memory_space=pltpu.SMEM),
        ],
        out_shape=[
            jax.ShapeDtypeStruct((PTC, 1), jnp.float32),
            jax.ShapeDtypeStruct((PTC, 1), jnp.int32),
            jax.ShapeDtypeStruct((1, 1), jnp.float32),
        ],
        scratch_shapes=[
            pltpu.VMEM((PB, 1), jnp.float32),
            pltpu.VMEM((PB, 1), jnp.int32),
            pltpu.SMEM((1,), jnp.float32),
        ],
        compiler_params=pltpu.CompilerParams(
            dimension_semantics=("arbitrary", "arbitrary"),
        ),
        interpret=interpret,
    )(points_tc, abc9)


def kernel(verts, faces, points):
    abc_flat = _sc_gather(verts.reshape(-1), faces.reshape(-1))
    abc9 = abc_flat.reshape(9, F)
    dist_sc, assoc_sc = _sc_sweep(abc_flat, points[:PSC].reshape(-1))
    dist2, assoc2, sum2 = _tc_sweep(points[PSC:], abc9)
    dist = jnp.concatenate([dist_sc, dist2.reshape(-1)])
    assoc = jnp.concatenate([assoc_sc, assoc2.reshape(-1)])
    loss = (sum2[0, 0] + jnp.sum(dist_sc)) * (1.0 / P)
    return loss, dist, assoc


# speedup vs baseline: 4.4987x; 1.5252x over previous
import functools

import jax
import jax.numpy as jnp
from jax import lax
from jax.experimental import pallas as pl
from jax.experimental.pallas import tpu as pltpu
from jax.experimental.pallas import tpu_sc as plsc

V, F, P = 4096, 8192, 8192
NW = 32
FPW = F // NW
L = 16

PSC = 1536
PTC = P - PSC

PB = 512
FB = 512
NPB = PTC // PB
NFB = F // FB

PPW = PSC // NW
INT_MAX = 2**31 - 1


def _safe_div(n, d):
    return n / jnp.where(jnp.abs(d) < 1e-12, 1.0, d)


def _tri_sq(px, py, pz, ax, ay, az, bx, by, bz, cx, cy, cz):
    abx = bx - ax
    aby = by - ay
    abz = bz - az
    acx = cx - ax
    acy = cy - ay
    acz = cz - az

    apx = px - ax
    apy = py - ay
    apz = pz - az
    d1 = abx * apx + aby * apy + abz * apz
    d2 = acx * apx + acy * apy + acz * apz
    bpx = px - bx
    bpy = py - by
    bpz = pz - bz
    d3 = abx * bpx + aby * bpy + abz * bpz
    d4 = acx * bpx + acy * bpy + acz * bpz
    cpx = px - cx
    cpy = py - cy
    cpz = pz - cz
    d5 = abx * cpx + aby * cpy + abz * cpz
    d6 = acx * cpx + acy * cpy + acz * cpz

    vc = d1 * d4 - d3 * d2
    vb = d5 * d2 - d1 * d6
    va = d3 * d6 - d5 * d4
    v_ab = _safe_div(d1, d1 - d3)
    w_ac = _safe_div(d2, d2 - d6)
    d43 = d4 - d3
    d56 = d5 - d6
    w_bc = _safe_div(d43, d43 + d56)
    denom = _safe_div(jnp.ones_like(va), va + vb + vc)
    v_in = vb * denom
    w_in = vc * denom

    clx = ax + abx * v_in + acx * w_in
    cly = ay + aby * v_in + acy * w_in
    clz = az + abz * v_in + acz * w_in

    m_bc = (va <= 0) & (d43 >= 0) & (d56 >= 0)
    clx = jnp.where(m_bc, bx + (cx - bx) * w_bc, clx)
    cly = jnp.where(m_bc, by + (cy - by) * w_bc, cly)
    clz = jnp.where(m_bc, bz + (cz - bz) * w_bc, clz)
    m_ac = (vb <= 0) & (d2 >= 0) & (d6 <= 0)
    clx = jnp.where(m_ac, ax + acx * w_ac, clx)
    cly = jnp.where(m_ac, ay + acy * w_ac, cly)
    clz = jnp.where(m_ac, az + acz * w_ac, clz)
    m_ab = (vc <= 0) & (d1 >= 0) & (d3 <= 0)
    clx = jnp.where(m_ab, ax + abx * v_ab, clx)
    cly = jnp.where(m_ab, ay + aby * v_ab, cly)
    clz = jnp.where(m_ab, az + abz * v_ab, clz)
    m_c = (d6 >= 0) & (d5 <= d6)
    clx = jnp.where(m_c, cx, clx)
    cly = jnp.where(m_c, cy, cly)
    clz = jnp.where(m_c, cz, clz)
    m_b = (d3 >= 0) & (d4 <= d3)
    clx = jnp.where(m_b, bx, clx)
    cly = jnp.where(m_b, by, cly)
    clz = jnp.where(m_b, bz, clz)
    m_a = (d1 <= 0) & (d2 <= 0)
    clx = jnp.where(m_a, ax, clx)
    cly = jnp.where(m_a, ay, cly)
    clz = jnp.where(m_a, az, clz)

    dx = px - clx
    dy = py - cly
    dz = pz - clz
    return dx * dx + dy * dy + dz * dz


def _sc_gather_body(verts_hbm, faces_hbm, out_hbm, verts_v, fidx_v, out_v):
    c = lax.axis_index("c")
    s = lax.axis_index("s")
    w = s * 2 + c
    base = w * FPW
    pltpu.sync_copy(verts_hbm, verts_v)
    pltpu.sync_copy(faces_hbm.at[pl.ds(3 * base, 3 * FPW)], fidx_v)
    iota3 = lax.iota(jnp.int32, L) * 3
    for i in range(FPW // L):
        for slot in range(3):
            r = plsc.load_gather(fidx_v, [iota3 + (48 * i + slot)])
            fi = r * 3
            for k in range(3):
                val = plsc.load_gather(verts_v, [fi + k])
                out_v[pl.ds((3 * slot + k) * FPW + i * L, L)] = val
    for row in range(9):
        pltpu.sync_copy(out_v.at[pl.ds(row * FPW, FPW)],
                        out_hbm.at[pl.ds(row * F + base, FPW)])


def _sc_gather(verts_flat, faces_flat):
    mesh = plsc.VectorSubcoreMesh(core_axis_name="c", subcore_axis_name="s")
    fn = pl.kernel(
        _sc_gather_body,
        out_type=jax.ShapeDtypeStruct((9 * F,), jnp.float32),
        mesh=mesh,
        compiler_params=pltpu.CompilerParams(needs_layout_passes=False),
        scratch_types=[
            pltpu.VMEM((3 * V,), jnp.float32),
            pltpu.VMEM((3 * FPW,), jnp.int32),
            pltpu.VMEM((9 * FPW,), jnp.float32),
        ],
    )
    return fn(verts_flat, faces_flat)


def _sc_sweep_body(abc_hbm, pts_hbm, dist_hbm, idx_hbm,
                   abc_v, pts_v, dist_v, idx_v):
    c = lax.axis_index("c")
    s = lax.axis_index("s")
    w = s * 2 + c
    pltpu.sync_copy(abc_hbm, abc_v)
    pltpu.sync_copy(pts_hbm.at[pl.ds(w * PPW * 3, PPW * 3)], pts_v)

    iota3 = lax.iota(jnp.int32, L) * 3

    def point_body(g, _):
        pbase = g * (L * 3)
        px = plsc.load_gather(pts_v, [iota3 + (pbase + 0)])
        py = plsc.load_gather(pts_v, [iota3 + (pbase + 1)])
        pz = plsc.load_gather(pts_v, [iota3 + (pbase + 2)])

        def blk_body(fb, carry):
            del fb
            rmin, ridx, bvec = carry
            for j in range(L):
                comps = [plsc.load_gather(abc_v, [bvec + (r * F + j)])
                         for r in range(9)]
                sq = _tri_sq(px, py, pz, *comps)
                better = sq < rmin
                rmin = jnp.where(better, sq, rmin)
                ridx = jnp.where(better, bvec + j, ridx)
            return rmin, ridx, bvec + L

        rmin0 = jnp.full((L,), jnp.inf, jnp.float32)
        ridx0 = jnp.zeros((L,), jnp.int32)
        bvec0 = jnp.zeros((L,), jnp.int32)
        rmin, ridx, _ = lax.fori_loop(0, F // L, blk_body,
                                      (rmin0, ridx0, bvec0))
        dist_v[pl.ds(g * L, L)] = rmin
        idx_v[pl.ds(g * L, L)] = ridx
        return 0

    lax.fori_loop(0, PPW // L, point_body, 0)

    pltpu.sync_copy(dist_v, dist_hbm.at[pl.ds(w * PPW, PPW)])
    pltpu.sync_copy(idx_v, idx_hbm.at[pl.ds(w * PPW, PPW)])


def _sc_sweep(abc_flat, pts_sc_flat):
    mesh = plsc.VectorSubcoreMesh(core_axis_name="c", subcore_axis_name="s")
    fn = pl.kernel(
        _sc_sweep_body,
        out_type=(jax.ShapeDtypeStruct((PSC,), jnp.float32),
                  jax.ShapeDtypeStruct((PSC,), jnp.int32)),
        mesh=mesh,
        compiler_params=pltpu.CompilerParams(needs_layout_passes=False),
        scratch_types=[
            pltpu.VMEM((9 * F,), jnp.float32),
            pltpu.VMEM((PPW * 3,), jnp.float32),
            pltpu.VMEM((PPW,), jnp.float32),
            pltpu.VMEM((PPW,), jnp.int32),
        ],
    )
    return fn(abc_flat, pts_sc_flat)


def _tc_body(pts_ref, abc_ref, dist_ref, assoc_ref, sum_ref,
             rmin_ref, ridx_ref, acc_ref):
    pi = pl.program_id(0)
    fj = pl.program_id(1)

    @pl.when(fj == 0)
    def _():
        rmin_ref[...] = jnp.full((PB, 1), jnp.inf, jnp.float32)
        ridx_ref[...] = jnp.zeros((PB, 1), jnp.int32)

    pts = pts_ref[...]
    px = pts[:, 0:1]
    py = pts[:, 1:2]
    pz = pts[:, 2:3]

    comps = [abc_ref[r:r + 1, :] for r in range(9)]
    sq = _tri_sq(px, py, pz, *comps)

    bmin = jnp.min(sq, axis=1, keepdims=True)
    lane = lax.broadcasted_iota(jnp.int32, (PB, FB), 1) + fj * FB
    cand = jnp.min(jnp.where(sq == bmin, lane, jnp.int32(INT_MAX)),
                   axis=1, keepdims=True)
    better = bmin < rmin_ref[...]
    rmin_ref[...] = jnp.where(better, bmin, rmin_ref[...])
    ridx_ref[...] = jnp.where(better, cand, ridx_ref[...])

    @pl.when(fj == NFB - 1)
    def _():
        dist_ref[...] = rmin_ref[...]
        assoc_ref[...] = ridx_ref[...]

        @pl.when(pi == 0)
        def _():
            acc_ref[0] = 0.0

        acc_ref[0] += jnp.sum(rmin_ref[...])

        @pl.when(pi == NPB - 1)
        def _():
            sum_ref[0, 0] = acc_ref[0]


def _tc_sweep(points_tc, abc9, interpret=False):
    return pl.pallas_call(
        _tc_body,
        grid=(NPB, NFB),
        in_specs=[
            pl.BlockSpec((PB, 3), lambda i, j: (i, 0)),
            pl.BlockSpec((9, FB), lambda i, j: (0, j)),
        ],
        out_specs=[
            pl.BlockSpec((PB, 1), lambda i, j: (i, 0)),
            pl.BlockSpec((PB, 1), lambda i, j: (i, 0)),
            pl.BlockSpec(memory_space=pltpu.SMEM),
        ],
        out_shape=[
            jax.ShapeDtypeStruct((PTC, 1), jnp.float32),
            jax.ShapeDtypeStruct((PTC, 1), jnp.int32),
            jax.ShapeDtypeStruct((1, 1), jnp.float32),
        ],
        scratch_shapes=[
            pltpu.VMEM((PB, 1), jnp.float32),
            pltpu.VMEM((PB, 1), jnp.int32),
            pltpu.SMEM((1,), jnp.float32),
        ],
        compiler_params=pltpu.CompilerParams(
            dimension_semantics=("arbitrary", "arbitrary"),
        ),
        interpret=interpret,
    )(points_tc, abc9)


def kernel(verts, faces, points):
    abc_flat = _sc_gather(verts.reshape(-1), faces.reshape(-1))
    abc9 = abc_flat.reshape(9, F)
    dist_sc, assoc_sc = _sc_sweep(abc_flat, points[:PSC].reshape(-1))
    dist2, assoc2, sum2 = _tc_sweep(points[PSC:], abc9)
    dist = jnp.concatenate([dist_sc, dist2.reshape(-1)])
    assoc = jnp.concatenate([assoc_sc, assoc2.reshape(-1)])
    loss = (sum2[0, 0] + jnp.sum(dist_sc)) * (1.0 / P)
    return loss, dist, assoc
